# baseline (speedup 1.0000x reference)
import functools

import jax
import jax.numpy as jnp
from jax import lax
from jax.experimental import pallas as pl
from jax.experimental.pallas import tpu as pltpu
from jax.experimental.pallas import tpu_sc as plsc

_E = 8
_CAP = 320
_S = 2048
_D = 1024
_F = 2048
_B = 2
_TOT = _B * _E * _CAP
_NW = 32
_SLOTS_W = _TOT // _NW
_TOK_W = (_B * _S) // _NW


def _router_body(x_ref, gw_ref, logits_ref, pmax_ref, eidx_ref,
                 gidx_ref, psl_ref, src_ref, init_ref):
    b = pl.program_id(0)
    x = x_ref[0]
    gw = gw_ref[...]
    l = jax.lax.dot_general(x, gw, (((1,), (1,)), ((), ())),
                            preferred_element_type=jnp.float32)
    logits_ref[0] = l
    m = jnp.max(l, axis=0, keepdims=True)
    u = jnp.exp(l - m)
    z = jnp.sum(u, axis=0, keepdims=True)
    probs = u / z
    best = probs[:, 0:1]
    te_f = jnp.zeros((_S, 1), jnp.float32)
    for e in range(1, _E):
        pe = probs[:, e:e + 1]
        gt = pe > best
        te_f = jnp.where(gt, jnp.float32(e), te_f)
        best = jnp.where(gt, pe, best)
    pmax_ref[0] = best
    iota_e = jax.lax.broadcasted_iota(jnp.int32, (_S, _E), 1).astype(
        jnp.float32)
    oh = (iota_e == te_f).astype(jnp.float32)
    r = jax.lax.broadcasted_iota(jnp.int32, (128, 128), 0)
    c = jax.lax.broadcasted_iota(jnp.int32, (128, 128), 1)
    tri = (r >= c).astype(jnp.float32)
    eye = (r == c).astype(jnp.float32)
    carry = jnp.zeros((1, _E), jnp.float32)
    sel_cols = []
    for k in range(_S // 128):
        sl = slice(k * 128, (k + 1) * 128)
        blk = oh[sl, :]
        pb = jax.lax.dot_general(tri, blk, (((1,), (0,)), ((), ())),
                                 precision=jax.lax.Precision.HIGHEST,
                                 preferred_element_type=jnp.float32) + carry
        carry = pb[127:128, :]
        sel_cols.append(jnp.sum(blk * pb, axis=1, keepdims=True))
    prio_sel = jnp.concatenate(sel_cols, axis=0)
    keep = (prio_sel <= _CAP).astype(jnp.float32)
    eidx_ref[0] = (te_f * keep).astype(jnp.int32)
    init_ref[0] = best * ((1.0 - keep) * x)

    _CH = 512
    cio = (jax.lax.broadcasted_iota(jnp.int32, (_CH, _CAP), 1) + 1
           ).astype(jnp.float32)
    sglob = (jax.lax.broadcasted_iota(jnp.int32, (_CH, 1), 0)
             ).astype(jnp.float32) + jnp.float32(_S) * b.astype(jnp.float32)
    gcols = [jnp.zeros((_CAP, 1), jnp.float32) for _ in range(_E)]
    pcols = [jnp.zeros((_CAP, 1), jnp.float32) for _ in range(_E)]
    for k in range(_S // _CH):
        sl = slice(k * _CH, (k + 1) * _CH)
        te_c = te_f[sl, :]
        ps_c = prio_sel[sl, :]
        sg_c = sglob + jnp.float32(k * _CH)
        best_c = best[sl, :]
        for e in range(_E):
            m_e = ((te_c == e) & (ps_c == cio)).astype(jnp.float32)
            gcols[e] = gcols[e] + jax.lax.dot_general(
                m_e, sg_c, (((0,), (0,)), ((), ())),
                precision=jax.lax.Precision.HIGHEST,
                preferred_element_type=jnp.float32)
            pcols[e] = pcols[e] + jax.lax.dot_general(
                m_e, best_c, (((0,), (0,)), ((), ())),
                precision=jax.lax.Precision.HIGHEST,
                preferred_element_type=jnp.float32)
    gidx_col = jnp.concatenate(gcols, axis=0)
    psl_col = jnp.concatenate(pcols, axis=0)

    slot_col = ((b.astype(jnp.float32) * _E + te_f) * _CAP
                + prio_sel - 1.0)
    tok_col = (jax.lax.broadcasted_iota(jnp.int32, (_S, 1), 0)
               ).astype(jnp.float32) + jnp.float32(_S) * b.astype(jnp.float32)
    src_col = jnp.where(keep > 0.0, slot_col, jnp.float32(_TOT) + tok_col)

    def _to_row(col):
        n = col.shape[0]
        return jnp.concatenate(
            [jax.lax.dot_general(col[j * 128:(j + 1) * 128, :], eye,
                                 (((0,), (0,)), ((), ())),
                                 precision=jax.lax.Precision.HIGHEST,
                                 preferred_element_type=jnp.float32)
             for j in range(n // 128)], axis=1)

    gidx_ref[0] = _to_row(gidx_col).astype(jnp.int32)
    psl_ref[0] = _to_row(psl_col)
    src_ref[0] = _to_row(src_col).astype(jnp.int32)


def _mlp_body(xe_ref, w1_ref, w2_ref, psl_ref, y_ref):
    xe = xe_ref[0, 0]
    h = jnp.maximum(
        jax.lax.dot_general(xe, w1_ref[0], (((1,), (0,)), ((), ())),
                            preferred_element_type=jnp.float32), 0.0)
    y = jax.lax.dot_general(h, w2_ref[0], (((1,), (0,)), ((), ())),
                            preferred_element_type=jnp.float32)
    y_ref[0, 0] = psl_ref[0, 0] * y


_mesh = plsc.VectorSubcoreMesh(core_axis_name="c", subcore_axis_name="s")


@functools.partial(
    pl.kernel,
    out_type=jax.ShapeDtypeStruct((_TOT, _D), jnp.float32),
    mesh=_mesh,
    scratch_types=[
        pltpu.VMEM((_SLOTS_W // 2,), jnp.int32),
        pltpu.VMEM((_SLOTS_W // 2,), jnp.int32),
        pltpu.VMEM((_SLOTS_W // 2, _D), jnp.float32),
        pltpu.SemaphoreType.DMA,
    ],
)
def _sc_gather(x_hbm, gidx_hbm, xe_hbm, idx_a, idx_b, rows, sem):
    wid = lax.axis_index("s") * 2 + lax.axis_index("c")
    base = wid * _SLOTS_W
    half = _SLOTS_W // 2
    pltpu.sync_copy(gidx_hbm.at[pl.ds(base, half)], idx_a)
    pltpu.sync_copy(gidx_hbm.at[pl.ds(base + half, half)], idx_b)
    pltpu.async_copy(x_hbm.at[idx_a], rows, sem).wait()
    pltpu.sync_copy(rows, xe_hbm.at[pl.ds(base, half)])
    pltpu.async_copy(x_hbm.at[idx_b], rows, sem).wait()
    pltpu.sync_copy(rows, xe_hbm.at[pl.ds(base + half, half)])


@functools.partial(
    pl.kernel,
    out_type=jax.ShapeDtypeStruct((_B * _S, _D), jnp.float32),
    mesh=_mesh,
    scratch_types=[
        pltpu.VMEM((_TOK_W // 2,), jnp.int32),
        pltpu.VMEM((_TOK_W // 2,), jnp.int32),
        pltpu.VMEM((_TOK_W // 2, _D), jnp.float32),
        pltpu.SemaphoreType.DMA,
    ],
)
def _sc_combine(src_hbm, sidx_hbm, out_hbm, idx_a, idx_b, rows, sem):
    wid = lax.axis_index("s") * 2 + lax.axis_index("c")
    base = wid * _TOK_W
    half = _TOK_W // 2
    pltpu.sync_copy(sidx_hbm.at[pl.ds(base, half)], idx_a)
    pltpu.sync_copy(sidx_hbm.at[pl.ds(base + half, half)], idx_b)
    pltpu.async_copy(src_hbm.at[idx_a], rows, sem).wait()
    pltpu.sync_copy(rows, out_hbm.at[pl.ds(base, half)])
    pltpu.async_copy(src_hbm.at[idx_b], rows, sem).wait()
    pltpu.sync_copy(rows, out_hbm.at[pl.ds(base + half, half)])


def kernel(norm_data, gate_w, W1, W2):
    f32 = jnp.float32
    i32 = jnp.int32
    (logits, pmax, eidx, gidx, psl, srcidx, out_init) = pl.pallas_call(
        _router_body,
        grid=(_B,),
        in_specs=[
            pl.BlockSpec((1, _S, _D), lambda b: (b, 0, 0)),
            pl.BlockSpec((_E, _D), lambda b: (0, 0)),
        ],
        out_specs=[
            pl.BlockSpec((1, _S, _E), lambda b: (b, 0, 0)),
            pl.BlockSpec((1, _S, 1), lambda b: (b, 0, 0)),
            pl.BlockSpec((1, _S, 1), lambda b: (b, 0, 0)),
            pl.BlockSpec((1, 1, _E * _CAP), lambda b: (b, 0, 0)),
            pl.BlockSpec((1, 1, _E * _CAP), lambda b: (b, 0, 0)),
            pl.BlockSpec((1, 1, _S), lambda b: (b, 0, 0)),
            pl.BlockSpec((1, _S, _D), lambda b: (b, 0, 0)),
        ],
        out_shape=[
            jax.ShapeDtypeStruct((_B, _S, _E), f32),
            jax.ShapeDtypeStruct((_B, _S, 1), f32),
            jax.ShapeDtypeStruct((_B, _S, 1), i32),
            jax.ShapeDtypeStruct((_B, 1, _E * _CAP), i32),
            jax.ShapeDtypeStruct((_B, 1, _E * _CAP), f32),
            jax.ShapeDtypeStruct((_B, 1, _S), i32),
            jax.ShapeDtypeStruct((_B, _S, _D), f32),
        ],
    )(norm_data, gate_w)

    x_flat = norm_data.reshape(_B * _S, _D)
    xe_all = _sc_gather(x_flat, gidx.reshape(_TOT))

    y = pl.pallas_call(
        _mlp_body,
        grid=(_E, _B),
        in_specs=[
            pl.BlockSpec((1, 1, _CAP, _D), lambda e, b: (b, e, 0, 0)),
            pl.BlockSpec((1, _D, _F), lambda e, b: (e, 0, 0)),
            pl.BlockSpec((1, _F, _D), lambda e, b: (e, 0, 0)),
            pl.BlockSpec((1, 1, _CAP, 1), lambda e, b: (b, e, 0, 0)),
        ],
        out_specs=pl.BlockSpec((1, 1, _CAP, _D), lambda e, b: (b, e, 0, 0)),
        out_shape=jax.ShapeDtypeStruct((_B, _E, _CAP, _D), f32),
        compiler_params=pltpu.CompilerParams(
            dimension_semantics=("arbitrary", "arbitrary")),
    )(xe_all.reshape(_B, _E, _CAP, _D), W1, W2,
      psl.reshape(_B, _E, _CAP, 1))

    src = jnp.concatenate(
        [y.reshape(_TOT, _D), out_init.reshape(_B * _S, _D)], axis=0)
    hidden = _sc_combine(src, srcidx.reshape(_B * _S)).reshape(_B, _S, _D)

    return hidden, logits, eidx.reshape(_B, _S)


# speedup vs baseline: 1.6436x; 1.6436x over previous
import jax
import jax.numpy as jnp
from jax.experimental import pallas as pl
from jax.experimental.pallas import tpu as pltpu

_E = 8
_CAP = 320
_S = 2048
_D = 1024
_F = 2048
_B = 2
_FSPLIT = 4
_FBLK = _F // _FSPLIT


def _router_body(x_ref, gw_ref, logits_ref, pmax_ref, keep_ref, eidx_ref,
                 terow_ref, psrow_ref):
    x = x_ref[0]
    gw = gw_ref[...]
    l = jax.lax.dot_general(x, gw, (((1,), (1,)), ((), ())),
                            preferred_element_type=jnp.float32)
    logits_ref[0] = l
    m = jnp.max(l, axis=0, keepdims=True)
    u = jnp.exp(l - m)
    z = jnp.sum(u, axis=0, keepdims=True)
    probs = u / z
    best = probs[:, 0:1]
    te_f = jnp.zeros((_S, 1), jnp.float32)
    for e in range(1, _E):
        pe = probs[:, e:e + 1]
        gt = pe > best
        te_f = jnp.where(gt, jnp.float32(e), te_f)
        best = jnp.where(gt, pe, best)
    pmax_ref[0] = best
    iota_e = jax.lax.broadcasted_iota(jnp.int32, (_S, _E), 1).astype(
        jnp.float32)
    oh = (iota_e == te_f).astype(jnp.float32)
    r = jax.lax.broadcasted_iota(jnp.int32, (128, 128), 0)
    c = jax.lax.broadcasted_iota(jnp.int32, (128, 128), 1)
    tri = (r >= c).astype(jnp.float32)
    eye = (r == c).astype(jnp.float32)
    carry = jnp.zeros((1, _E), jnp.float32)
    sel_cols = []
    te_rows = []
    ps_rows = []
    for k in range(_S // 128):
        sl = slice(k * 128, (k + 1) * 128)
        blk = oh[sl, :]
        pb = jax.lax.dot_general(tri, blk, (((1,), (0,)), ((), ())),
                                 precision=jax.lax.Precision.HIGHEST,
                                 preferred_element_type=jnp.float32) + carry
        carry = pb[127:128, :]
        sel_blk = jnp.sum(blk * pb, axis=1, keepdims=True)
        sel_cols.append(sel_blk)
        te_rows.append(jax.lax.dot_general(
            te_f[sl, :], eye, (((0,), (0,)), ((), ())),
            precision=jax.lax.Precision.HIGHEST,
            preferred_element_type=jnp.float32))
        ps_rows.append(jax.lax.dot_general(
            sel_blk, eye, (((0,), (0,)), ((), ())),
            precision=jax.lax.Precision.HIGHEST,
            preferred_element_type=jnp.float32))
    prio_sel = jnp.concatenate(sel_cols, axis=0)
    keep = (prio_sel <= _CAP).astype(jnp.float32)
    keep_ref[0] = keep
    eidx_ref[0] = (te_f * keep).astype(jnp.int32)
    terow_ref[0] = jnp.concatenate(te_rows, axis=1)
    psrow_ref[0] = jnp.concatenate(ps_rows, axis=1)


def _expert_body(x_ref, w1_ref, w2_ref, terow_ref, psrow_ref, pmax_ref,
                 keep_ref, out_ref, msk_ref, xe_ref, y_ref):
    e = pl.program_id(1)
    f = pl.program_id(2)

    @pl.when(f == 0)
    def _():
        te_row = terow_ref[0]
        ps_row = psrow_ref[0]
        cio = (jax.lax.broadcasted_iota(jnp.int32, (_CAP, _S), 0) + 1
               ).astype(jnp.float32)
        msk_ref[...] = ((te_row == e.astype(jnp.float32)) & (ps_row == cio)
                        ).astype(jnp.float32)
        xe_ref[...] = jax.lax.dot_general(
            msk_ref[...], x_ref[0], (((1,), (0,)), ((), ())),
            preferred_element_type=jnp.float32)

    h = jnp.maximum(
        jax.lax.dot_general(xe_ref[...], w1_ref[0], (((1,), (0,)), ((), ())),
                            preferred_element_type=jnp.float32), 0.0)
    yp = jax.lax.dot_general(h, w2_ref[0], (((1,), (0,)), ((), ())),
                             preferred_element_type=jnp.float32)

    @pl.when(f == 0)
    def _():
        y_ref[...] = yp

    @pl.when(f > 0)
    def _():
        y_ref[...] = y_ref[...] + yp

    @pl.when((e == 0) & (f == 0))
    def _():
        out_ref[0] = (1.0 - keep_ref[0]) * x_ref[0]

    @pl.when(f == _FSPLIT - 1)
    def _():
        out_ref[0] = out_ref[0] + jax.lax.dot_general(
            msk_ref[...], y_ref[...], (((0,), (0,)), ((), ())),
            preferred_element_type=jnp.float32)

    @pl.when((e == _E - 1) & (f == _FSPLIT - 1))
    def _():
        out_ref[0] = pmax_ref[0] * out_ref[0]


def kernel(norm_data, gate_w, W1, W2):
    f32 = jnp.float32
    i32 = jnp.int32
    logits, pmax, keep, eidx, te_row, ps_row = pl.pallas_call(
        _router_body,
        grid=(_B,),
        in_specs=[
            pl.BlockSpec((1, _S, _D), lambda b: (b, 0, 0)),
            pl.BlockSpec((_E, _D), lambda b: (0, 0)),
        ],
        out_specs=[
            pl.BlockSpec((1, _S, _E), lambda b: (b, 0, 0)),
            pl.BlockSpec((1, _S, 1), lambda b: (b, 0, 0)),
            pl.BlockSpec((1, _S, 1), lambda b: (b, 0, 0)),
            pl.BlockSpec((1, _S, 1), lambda b: (b, 0, 0)),
            pl.BlockSpec((1, 1, _S), lambda b: (b, 0, 0)),
            pl.BlockSpec((1, 1, _S), lambda b: (b, 0, 0)),
        ],
        out_shape=[
            jax.ShapeDtypeStruct((_B, _S, _E), f32),
            jax.ShapeDtypeStruct((_B, _S, 1), f32),
            jax.ShapeDtypeStruct((_B, _S, 1), f32),
            jax.ShapeDtypeStruct((_B, _S, 1), i32),
            jax.ShapeDtypeStruct((_B, 1, _S), f32),
            jax.ShapeDtypeStruct((_B, 1, _S), f32),
        ],
    )(norm_data, gate_w)

    hidden = pl.pallas_call(
        _expert_body,
        grid=(_B, _E, _FSPLIT),
        in_specs=[
            pl.BlockSpec((1, _S, _D), lambda b, e, f: (b, 0, 0)),
            pl.BlockSpec((1, _D, _FBLK), lambda b, e, f: (e, 0, f)),
            pl.BlockSpec((1, _FBLK, _D), lambda b, e, f: (e, f, 0)),
            pl.BlockSpec((1, 1, _S), lambda b, e, f: (b, 0, 0)),
            pl.BlockSpec((1, 1, _S), lambda b, e, f: (b, 0, 0)),
            pl.BlockSpec((1, _S, 1), lambda b, e, f: (b, 0, 0)),
            pl.BlockSpec((1, _S, 1), lambda b, e, f: (b, 0, 0)),
        ],
        out_specs=pl.BlockSpec((1, _S, _D), lambda b, e, f: (b, 0, 0)),
        out_shape=jax.ShapeDtypeStruct((_B, _S, _D), f32),
        scratch_shapes=[
            pltpu.VMEM((_CAP, _S), f32),
            pltpu.VMEM((_CAP, _D), f32),
            pltpu.VMEM((_CAP, _D), f32),
        ],
        compiler_params=pltpu.CompilerParams(
            dimension_semantics=("arbitrary", "arbitrary", "arbitrary")),
    )(norm_data, W1, W2, te_row, ps_row, pmax, keep)

    return hidden, logits, eidx.reshape(_B, _S)
